# baseline (speedup 1.0000x reference)
import functools

import jax
import jax.numpy as jnp
from jax import lax
from jax.experimental import pallas as pl
from jax.experimental.pallas import tpu as pltpu
from jax.experimental.pallas import tpu_sc as plsc


def _proj_body(x_ref, w_ref, b_ref, o_ref):
    o_ref[...] = (
        lax.dot_general(
            x_ref[...], w_ref[...], (((1,), (1,)), ((), ())),
            preferred_element_type=jnp.float32,
        )
        + b_ref[...]
    )


def _project_table(table, W, b, blk):
    V, E = table.shape
    P = W.shape[0]
    assert V % blk == 0
    return pl.pallas_call(
        _proj_body,
        grid=(V // blk,),
        in_specs=[
            pl.BlockSpec((blk, E), lambda i: (i, 0)),
            pl.BlockSpec((P, E), lambda i: (0, 0)),
            pl.BlockSpec((1, P), lambda i: (0, 0)),
        ],
        out_specs=pl.BlockSpec((blk, P), lambda i: (i, 0)),
        out_shape=jax.ShapeDtypeStruct((V, P), jnp.float32),
    )(table, W, b.reshape(1, P))


CHUNK = 64
NBUF = 10
LOOKAHEAD = 6


@functools.lru_cache(maxsize=None)
def _make_gather(V, D, n_chunks):
    info = plsc.get_sparse_core_info()
    nw = info.num_cores * info.num_subcores
    mesh = plsc.VectorSubcoreMesh(core_axis_name="c", subcore_axis_name="s")
    assert n_chunks % NBUF == 0 and n_chunks // NBUF >= 2

    @functools.partial(
        pl.kernel,
        mesh=mesh,
        out_type=jax.ShapeDtypeStruct((nw, n_chunks, CHUNK, D), jnp.float32),
        scratch_types=[
            pltpu.VMEM((n_chunks, CHUNK), jnp.int32),
            pltpu.VMEM((NBUF, CHUNK, D), jnp.float32),
        ]
        + [pltpu.SemaphoreType.DMA] * (2 * NBUF),
    )
    def gather_kernel(table_hbm, idx_hbm, out_hbm, idx_v, bufs, *sems):
        sem_in = sems[:NBUF]
        sem_out = sems[NBUF:]
        wid = lax.axis_index("s") * info.num_cores + lax.axis_index("c")
        pltpu.sync_copy(idx_hbm.at[wid], idx_v)

        def issue_gather(j, b):
            pltpu.async_copy(table_hbm.at[idx_v.at[j]], bufs.at[b], sem_in[b])

        def wait_gather(j, b):
            pltpu.make_async_copy(
                table_hbm.at[idx_v.at[j]], bufs.at[b], sem_in[b]
            ).wait()

        def issue_out(j, b):
            pltpu.async_copy(bufs.at[b], out_hbm.at[wid, j], sem_out[b])

        def wait_out(j, b):
            pltpu.make_async_copy(
                bufs.at[b], out_hbm.at[wid, j], sem_out[b]
            ).wait()

        def step(j, g, b):
            wait_gather(j, b)
            issue_out(j, b)
            jn = j + LOOKAHEAD
            bn = (b + LOOKAHEAD) % NBUF
            if g is not None:
                wait_out(jn, bn)
                issue_gather(jn, bn)

        for b in range(LOOKAHEAD):
            issue_gather(b, b)
        for b in range(NBUF):
            j = b
            wait_gather(j, b)
            issue_out(j, b)
            jn = j + LOOKAHEAD
            bn = (b + LOOKAHEAD) % NBUF
            if jn >= NBUF:
                wait_out(jn - NBUF, bn)
            issue_gather(jn, bn)

        def body(g, carry):
            for b in range(NBUF):
                step(g * NBUF + b, g, b)
            return carry

        lax.fori_loop(1, n_chunks // NBUF - 1, body, 0)

        for b in range(NBUF):
            j = n_chunks - NBUF + b
            wait_gather(j, b)
            issue_out(j, b)
            jn = j + LOOKAHEAD
            bn = (b + LOOKAHEAD) % NBUF
            if jn < n_chunks:
                wait_out(jn - NBUF, bn)
                issue_gather(jn, bn)

        for b in range(NBUF):
            j = n_chunks - NBUF + b
            wait_out(j, b)

    return gather_kernel, nw


def kernel(ext_word_ids, seq_lengths, embed_table, W, b):
    del seq_lengths
    Bsz, Lseq = ext_word_ids.shape
    V, E = embed_table.shape
    P = W.shape[0]

    proj = _project_table(embed_table, W, b, blk=25000)

    total = Bsz * Lseq
    nw = 32
    n_chunks = total // (nw * CHUNK)
    gather_fn, nw = _make_gather(V, P, n_chunks)
    ids = ext_word_ids.reshape(nw, n_chunks, CHUNK).astype(jnp.int32)
    out = gather_fn(proj, ids)
    return out.reshape(Bsz, Lseq, P)


# speedup vs baseline: 1.0167x; 1.0167x over previous
import functools

import jax
import jax.numpy as jnp
from jax import lax
from jax.experimental import pallas as pl
from jax.experimental.pallas import tpu as pltpu
from jax.experimental.pallas import tpu_sc as plsc


def _proj_body(x_ref, w_ref, b_ref, o_ref):
    o_ref[...] = (
        lax.dot_general(
            x_ref[...], w_ref[...], (((1,), (1,)), ((), ())),
            preferred_element_type=jnp.float32,
        )
        + b_ref[...]
    )


def _project_table(table, W, b, blk):
    V, E = table.shape
    P = W.shape[0]
    assert V % blk == 0
    return pl.pallas_call(
        _proj_body,
        grid=(V // blk,),
        in_specs=[
            pl.BlockSpec((blk, E), lambda i: (i, 0)),
            pl.BlockSpec((P, E), lambda i: (0, 0)),
            pl.BlockSpec((1, P), lambda i: (0, 0)),
        ],
        out_specs=pl.BlockSpec((blk, P), lambda i: (i, 0)),
        out_shape=jax.ShapeDtypeStruct((V, P), jnp.float32),
    )(table, W, b.reshape(1, P))


CHUNK = 64
NBUF = 10
LOOKAHEAD = 8


@functools.lru_cache(maxsize=None)
def _make_gather(V, D, n_chunks):
    info = plsc.get_sparse_core_info()
    nw = info.num_cores * info.num_subcores
    mesh = plsc.VectorSubcoreMesh(core_axis_name="c", subcore_axis_name="s")
    assert n_chunks % NBUF == 0 and n_chunks // NBUF >= 2

    @functools.partial(
        pl.kernel,
        mesh=mesh,
        out_type=jax.ShapeDtypeStruct((nw, n_chunks, CHUNK, D), jnp.float32),
        scratch_types=[
            pltpu.VMEM((n_chunks, CHUNK), jnp.int32),
            pltpu.VMEM((NBUF, CHUNK, D), jnp.float32),
        ]
        + [pltpu.SemaphoreType.DMA] * (2 * NBUF),
    )
    def gather_kernel(table_hbm, idx_hbm, out_hbm, idx_v, bufs, *sems):
        sem_in = sems[:NBUF]
        sem_out = sems[NBUF:]
        wid = lax.axis_index("s") * info.num_cores + lax.axis_index("c")
        pltpu.sync_copy(idx_hbm.at[wid], idx_v)

        def issue_gather(j, b):
            pltpu.async_copy(table_hbm.at[idx_v.at[j]], bufs.at[b], sem_in[b])

        def wait_gather(j, b):
            pltpu.make_async_copy(
                table_hbm.at[idx_v.at[j]], bufs.at[b], sem_in[b]
            ).wait()

        def issue_out(j, b):
            pltpu.async_copy(bufs.at[b], out_hbm.at[wid, j], sem_out[b])

        def wait_out(j, b):
            pltpu.make_async_copy(
                bufs.at[b], out_hbm.at[wid, j], sem_out[b]
            ).wait()

        def step(j, g, b):
            wait_gather(j, b)
            issue_out(j, b)
            jn = j + LOOKAHEAD
            bn = (b + LOOKAHEAD) % NBUF
            if g is not None:
                wait_out(jn, bn)
                issue_gather(jn, bn)

        for b in range(LOOKAHEAD):
            issue_gather(b, b)
        for b in range(NBUF):
            j = b
            wait_gather(j, b)
            issue_out(j, b)
            jn = j + LOOKAHEAD
            bn = (b + LOOKAHEAD) % NBUF
            if jn >= NBUF:
                wait_out(jn - NBUF, bn)
            issue_gather(jn, bn)

        def body(g, carry):
            for b in range(NBUF):
                step(g * NBUF + b, g, b)
            return carry

        lax.fori_loop(1, n_chunks // NBUF - 1, body, 0)

        for b in range(NBUF):
            j = n_chunks - NBUF + b
            wait_gather(j, b)
            issue_out(j, b)
            jn = j + LOOKAHEAD
            bn = (b + LOOKAHEAD) % NBUF
            if jn < n_chunks:
                wait_out(jn - NBUF, bn)
                issue_gather(jn, bn)

        for b in range(NBUF):
            j = n_chunks - NBUF + b
            wait_out(j, b)

    return gather_kernel, nw


def kernel(ext_word_ids, seq_lengths, embed_table, W, b):
    del seq_lengths
    Bsz, Lseq = ext_word_ids.shape
    V, E = embed_table.shape
    P = W.shape[0]

    proj = _project_table(embed_table, W, b, blk=20000)

    total = Bsz * Lseq
    nw = 32
    n_chunks = total // (nw * CHUNK)
    gather_fn, nw = _make_gather(V, P, n_chunks)
    ids = ext_word_ids.reshape(nw, n_chunks, CHUNK).astype(jnp.int32)
    out = gather_fn(proj, ids)
    return out.reshape(Bsz, Lseq, P)


# speedup vs baseline: 1.0179x; 1.0011x over previous
import functools

import jax
import jax.numpy as jnp
from jax import lax
from jax.experimental import pallas as pl
from jax.experimental.pallas import tpu as pltpu
from jax.experimental.pallas import tpu_sc as plsc


def _proj_body(x_ref, w_ref, b_ref, o_ref):
    o_ref[...] = (
        lax.dot_general(
            x_ref[...], w_ref[...], (((1,), (1,)), ((), ())),
            preferred_element_type=jnp.float32,
        )
        + b_ref[...]
    )


def _project_table(table, W, b, blk):
    V, E = table.shape
    P = W.shape[0]
    assert V % blk == 0
    return pl.pallas_call(
        _proj_body,
        grid=(V // blk,),
        in_specs=[
            pl.BlockSpec((blk, E), lambda i: (i, 0)),
            pl.BlockSpec((P, E), lambda i: (0, 0)),
            pl.BlockSpec((1, P), lambda i: (0, 0)),
        ],
        out_specs=pl.BlockSpec((blk, P), lambda i: (i, 0)),
        out_shape=jax.ShapeDtypeStruct((V, P), jnp.float32),
    )(table, W, b.reshape(1, P))


CHUNK = 64
NBUF = 10
LOOKAHEAD = 6


@functools.lru_cache(maxsize=None)
def _make_gather(V, D, n_chunks):
    info = plsc.get_sparse_core_info()
    nw = info.num_cores * info.num_subcores
    mesh = plsc.VectorSubcoreMesh(core_axis_name="c", subcore_axis_name="s")
    assert n_chunks % NBUF == 0 and n_chunks // NBUF >= 2

    @functools.partial(
        pl.kernel,
        mesh=mesh,
        out_type=jax.ShapeDtypeStruct((nw, n_chunks, CHUNK, D), jnp.float32),
        scratch_types=[
            pltpu.VMEM((n_chunks, CHUNK), jnp.int32),
            pltpu.VMEM((NBUF, CHUNK, D), jnp.float32),
        ]
        + [pltpu.SemaphoreType.DMA] * (2 * NBUF),
    )
    def gather_kernel(table_hbm, idx_hbm, out_hbm, idx_v, bufs, *sems):
        sem_in = sems[:NBUF]
        sem_out = sems[NBUF:]
        wid = lax.axis_index("s") * info.num_cores + lax.axis_index("c")
        pltpu.sync_copy(idx_hbm.at[wid], idx_v)

        def issue_gather(j, b):
            pltpu.async_copy(table_hbm.at[idx_v.at[j]], bufs.at[b], sem_in[b])

        def wait_gather(j, b):
            pltpu.make_async_copy(
                table_hbm.at[idx_v.at[j]], bufs.at[b], sem_in[b]
            ).wait()

        def issue_out(j, b):
            pltpu.async_copy(bufs.at[b], out_hbm.at[wid, j], sem_out[b])

        def wait_out(j, b):
            pltpu.make_async_copy(
                bufs.at[b], out_hbm.at[wid, j], sem_out[b]
            ).wait()

        def step(j, g, b):
            wait_gather(j, b)
            issue_out(j, b)
            jn = j + LOOKAHEAD
            bn = (b + LOOKAHEAD) % NBUF
            if g is not None:
                wait_out(jn, bn)
                issue_gather(jn, bn)

        for b in range(LOOKAHEAD):
            issue_gather(b, b)
        for b in range(NBUF):
            j = b
            wait_gather(j, b)
            issue_out(j, b)
            jn = j + LOOKAHEAD
            bn = (b + LOOKAHEAD) % NBUF
            if jn >= NBUF:
                wait_out(jn - NBUF, bn)
            issue_gather(jn, bn)

        def body(g, carry):
            for b in range(NBUF):
                step(g * NBUF + b, g, b)
            return carry

        lax.fori_loop(1, n_chunks // NBUF - 1, body, 0)

        for b in range(NBUF):
            j = n_chunks - NBUF + b
            wait_gather(j, b)
            issue_out(j, b)
            jn = j + LOOKAHEAD
            bn = (b + LOOKAHEAD) % NBUF
            if jn < n_chunks:
                wait_out(jn - NBUF, bn)
                issue_gather(jn, bn)

        for b in range(NBUF):
            j = n_chunks - NBUF + b
            wait_out(j, b)

    return gather_kernel, nw


def kernel(ext_word_ids, seq_lengths, embed_table, W, b):
    del seq_lengths
    Bsz, Lseq = ext_word_ids.shape
    V, E = embed_table.shape
    P = W.shape[0]

    proj = _project_table(embed_table, W, b, blk=20000)

    total = Bsz * Lseq
    nw = 32
    n_chunks = total // (nw * CHUNK)
    gather_fn, nw = _make_gather(V, P, n_chunks)
    ids = ext_word_ids.reshape(nw, n_chunks, CHUNK).astype(jnp.int32)
    out = gather_fn(proj, ids)
    return out.reshape(Bsz, Lseq, P)
